# Initial kernel scaffold; baseline (speedup 1.0000x reference)
import jax
import jax.numpy as jnp
from jax.experimental import pallas as pl


def kernel(batch_u, batch_n, batch_v_pad, batch_v_mask, batch_vsp, W_i, W_u, W_v):
    raise NotImplementedError("write your pallas kernel here")


# speedup vs baseline: 7.7551x; 7.7551x over previous
import functools

import jax
import jax.numpy as jnp
from jax import lax
from jax.experimental import pallas as pl
from jax.experimental.pallas import tpu as pltpu
from jax.experimental.pallas import tpu_sc as plsc

_EMB_DIM = 128
_CTX = 50
_B = 16384
_L = 16
_NSTRIP = _EMB_DIM // _L
_NB = 16
_MARGIN = 1.0
_SIGMA = 0.01


def _sc_gather_scores(batch_u, batch_n, batch_v_pad, batch_v_mask, batch_vsp,
                      W_i, W_u, W_v):
    info = plsc.get_sparse_core_info()
    nc, ns = info.num_cores, info.num_subcores
    nw = nc * ns
    bpw = _B // nw
    nch = bpw // _NB

    mesh = plsc.VectorSubcoreMesh(core_axis_name="c", subcore_axis_name="s")

    @functools.partial(
        pl.kernel,
        mesh=mesh,
        out_type=[
            jax.ShapeDtypeStruct((_B, _L), jnp.float32),
            jax.ShapeDtypeStruct((_B, _L), jnp.float32),
            jax.ShapeDtypeStruct((nw, _L), jnp.float32),
        ],
        scratch_types=[
            pltpu.VMEM((_NB, _CTX), jnp.int32),
            pltpu.VMEM((_NB, _CTX), jnp.float32),
            pltpu.VMEM((_NB,), jnp.int32),
            pltpu.VMEM((_NB,), jnp.int32),
            pltpu.VMEM((_NB,), jnp.int32),
            pltpu.VMEM((_NB * _CTX, _EMB_DIM), jnp.float32),
            pltpu.VMEM((_NB, _EMB_DIM), jnp.float32),
            pltpu.VMEM((_NB, _EMB_DIM), jnp.float32),
            pltpu.VMEM((_NB, _EMB_DIM), jnp.float32),
            pltpu.VMEM((_NB, _EMB_DIM), jnp.float32),
            pltpu.VMEM((_NB, _L), jnp.float32),
            pltpu.VMEM((_NB, _L), jnp.float32),
            pltpu.VMEM((_L,), jnp.float32),
            pltpu.SemaphoreType.DMA,
        ],
    )
    def k(bu_h, bn_h, bvp_h, bvm_h, bvsp_h, wi_h, wu_h, wv_h,
          outc_h, outn_h, outvsp_h,
          idxv, maskv, ubuf, nbuf, vspbuf, rows, urows, nrows, vspu, vspi,
          outc_v, outn_v, vacc, sem):
        wid = lax.axis_index("s") * nc + lax.axis_index("c")
        vacc[...] = jnp.zeros((_L,), jnp.float32)

        def chunk_body(g, _):
            b0 = wid * bpw + g * _NB
            pltpu.sync_copy(bvp_h.at[pl.ds(b0, _NB)], idxv)
            pltpu.sync_copy(bvm_h.at[pl.ds(b0, _NB)], maskv)
            pltpu.sync_copy(bu_h.at[pl.ds(b0, _NB)], ubuf)
            pltpu.sync_copy(bn_h.at[pl.ds(b0, _NB)], nbuf)
            pltpu.sync_copy(bvsp_h.at[pl.ds(b0, _NB)], vspbuf)
            handles = []
            for b in range(_NB):
                handles.append(pltpu.async_copy(
                    wv_h.at[idxv.at[b]], rows.at[pl.ds(b * _CTX, _CTX)], sem))
            handles.append(pltpu.async_copy(wu_h.at[ubuf], urows, sem))
            handles.append(pltpu.async_copy(wu_h.at[nbuf], nrows, sem))
            handles.append(pltpu.async_copy(wu_h.at[vspbuf], vspu, sem))
            handles.append(pltpu.async_copy(wi_h.at[vspbuf], vspi, sem))
            for h in handles:
                h.wait()

            groups = [(0, 0, _L), (16, 0, _L), (32, 0, _L), (34, 14, 2)]

            def b_body(b, _):
                embv = [jnp.zeros((_L,), jnp.float32)] * _NSTRIP
                for start, lane0, nlanes in groups:
                    mv = maskv[b, pl.ds(start, _L)]
                    for l in range(lane0, lane0 + nlanes):
                        c = start + l
                        r = b * _CTX + c
                        m = mv[l]
                        for j in range(_NSTRIP):
                            embv[j] = embv[j] + m * rows[r, pl.ds(_L * j, _L)]

                pc = jnp.zeros((_L,), jnp.float32)
                pn = jnp.zeros((_L,), jnp.float32)
                sq = jnp.zeros((_L,), jnp.float32)
                for j in range(_NSTRIP):
                    sl = pl.ds(_L * j, _L)
                    pc = pc + urows[b, sl] * embv[j]
                    pn = pn + nrows[b, sl] * embv[j]
                    dd = vspu[b, sl] - vspi[b, sl]
                    sq = sq + dd * dd
                outc_v[b, :] = pc
                outn_v[b, :] = pn
                vacc[...] = vacc[...] + sq
                return 0

            lax.fori_loop(0, _NB, b_body, 0)

            pltpu.sync_copy(outc_v, outc_h.at[pl.ds(b0, _NB)])
            pltpu.sync_copy(outn_v, outn_h.at[pl.ds(b0, _NB)])
            return 0

        lax.fori_loop(0, nch, chunk_body, 0)
        pltpu.sync_copy(vacc, outvsp_h.at[wid])

    return k(batch_u, batch_n, batch_v_pad, batch_v_mask, batch_vsp,
             W_i, W_u, W_v)


def _tc_loss_body(pc_ref, pn_ref, vsp_ref, out_ref):
    sc = jnp.sum(pc_ref[...], axis=1)
    sn = jnp.sum(pn_ref[...], axis=1)

    def log_sigmoid(x):
        return jnp.minimum(x, 0.0) - jnp.log(1.0 + jnp.exp(-jnp.abs(x)))

    score1 = jnp.sum(jnp.maximum(_MARGIN + log_sigmoid(sn) - log_sigmoid(sc),
                                 0.0))
    out_ref[0, 0] = score1 + _SIGMA * jnp.sum(vsp_ref[...])


def kernel(batch_u, batch_n, batch_v_pad, batch_v_mask, batch_vsp,
           W_i, W_u, W_v):
    out_c, out_n, out_vsp = _sc_gather_scores(
        batch_u, batch_n, batch_v_pad, batch_v_mask, batch_vsp, W_i, W_u, W_v)
    loss = pl.pallas_call(
        _tc_loss_body,
        out_shape=jax.ShapeDtypeStruct((1, 1), jnp.float32),
        out_specs=pl.BlockSpec(memory_space=pltpu.SMEM),
    )(out_c, out_n, out_vsp)
    return loss[0, 0]


# speedup vs baseline: 9.2099x; 1.1876x over previous
import functools

import jax
import jax.numpy as jnp
from jax import lax
from jax.experimental import pallas as pl
from jax.experimental.pallas import tpu as pltpu
from jax.experimental.pallas import tpu_sc as plsc

_EMB_DIM = 128
_CTX = 50
_B = 16384
_L = 16
_NSTRIP = _EMB_DIM // _L
_NB = 8
_MARGIN = 1.0
_SIGMA = 0.01

_MASK_GROUPS = [(0, 0, 16), (16, 0, 16), (32, 0, 16), (34, 14, 2)]


def _sc_gather_scores(batch_u, batch_n, batch_v_pad, batch_v_mask, batch_vsp,
                      W_i, W_u, W_v):
    info = plsc.get_sparse_core_info()
    nc, ns = info.num_cores, info.num_subcores
    nw = nc * ns
    bpw = _B // nw
    nch = bpw // _NB

    mesh = plsc.VectorSubcoreMesh(core_axis_name="c", subcore_axis_name="s")

    buf_types = []
    for _ in range(2):
        buf_types += [
            pltpu.VMEM((_NB, _CTX), jnp.int32),
            pltpu.VMEM((_NB, _CTX), jnp.float32),
            pltpu.VMEM((_NB,), jnp.int32),
            pltpu.VMEM((_NB,), jnp.int32),
            pltpu.VMEM((_NB,), jnp.int32),
            pltpu.VMEM((_NB * _CTX, _EMB_DIM), jnp.float32),
            pltpu.VMEM((_NB, _EMB_DIM), jnp.float32),
            pltpu.VMEM((_NB, _EMB_DIM), jnp.float32),
            pltpu.VMEM((_NB, _EMB_DIM), jnp.float32),
            pltpu.VMEM((_NB, _EMB_DIM), jnp.float32),
            pltpu.SemaphoreType.DMA,
            pltpu.VMEM((_NB * _L,), jnp.float32),
            pltpu.VMEM((_NB * _L,), jnp.float32),
            pltpu.SemaphoreType.DMA,
        ]

    @functools.partial(
        pl.kernel,
        mesh=mesh,
        out_type=[
            jax.ShapeDtypeStruct((_B * _L,), jnp.float32),
            jax.ShapeDtypeStruct((_B * _L,), jnp.float32),
            jax.ShapeDtypeStruct((nw * _L,), jnp.float32),
        ],
        scratch_types=buf_types + [
            pltpu.VMEM((_L,), jnp.float32),
        ],
    )
    def k(bu_h, bn_h, bvp_h, bvm_h, bvsp_h, wi_h, wu_h, wv_h,
          outc_h, outn_h, outvsp_h, *scratch):
        bufs = (scratch[0:14], scratch[14:28])
        vacc = scratch[28]
        wid = lax.axis_index("s") * nc + lax.axis_index("c")
        vacc[...] = jnp.zeros((_L,), jnp.float32)

        def copies(g, p, issue):
            (idxv, maskv, ubuf, nbuf, vspbuf, rows,
             urows, nrows, vspu, vspi, sem, _, _, _) = bufs[p]
            b0 = wid * bpw + g * _NB
            if issue:
                pltpu.sync_copy(bvp_h.at[pl.ds(b0, _NB)], idxv)
                pltpu.sync_copy(bvm_h.at[pl.ds(b0, _NB)], maskv)
                pltpu.sync_copy(bu_h.at[pl.ds(b0, _NB)], ubuf)
                pltpu.sync_copy(bn_h.at[pl.ds(b0, _NB)], nbuf)
                pltpu.sync_copy(bvsp_h.at[pl.ds(b0, _NB)], vspbuf)
            descs = []
            for b in range(_NB):
                descs.append(pltpu.make_async_copy(
                    wv_h.at[idxv.at[b]], rows.at[pl.ds(b * _CTX, _CTX)], sem))
            descs.append(pltpu.make_async_copy(wu_h.at[ubuf], urows, sem))
            descs.append(pltpu.make_async_copy(wu_h.at[nbuf], nrows, sem))
            descs.append(pltpu.make_async_copy(wu_h.at[vspbuf], vspu, sem))
            descs.append(pltpu.make_async_copy(wi_h.at[vspbuf], vspi, sem))
            for d in descs:
                if issue:
                    d.start()
                else:
                    d.wait()

        def out_copies(g, p, issue):
            (_, _, _, _, _, _, _, _, _, _, _, outc_s, outn_s, osem) = bufs[p]
            b0 = wid * bpw + g * _NB
            for src, dst_h in ((outc_s, outc_h), (outn_s, outn_h)):
                d = pltpu.make_async_copy(
                    src, dst_h.at[pl.ds(b0 * _L, _NB * _L)], osem)
                if issue:
                    d.start()
                else:
                    d.wait()

        def compute(g, p):
            (idxv, maskv, ubuf, nbuf, vspbuf, rows,
             urows, nrows, vspu, vspi, sem, outc_s, outn_s, osem) = bufs[p]

            @pl.when(g >= 2)
            def _():
                out_copies(g - 2, p, issue=False)

            def b_body(b, _):
                embv = [jnp.zeros((_L,), jnp.float32)] * _NSTRIP
                for start, lane0, nlanes in _MASK_GROUPS:
                    mv = maskv[b, pl.ds(start, _L)]
                    for l in range(lane0, lane0 + nlanes):
                        c = start + l
                        r = b * _CTX + c
                        m = mv[l]
                        for j in range(_NSTRIP):
                            embv[j] = embv[j] + m * rows[r, pl.ds(_L * j, _L)]

                pc = jnp.zeros((_L,), jnp.float32)
                pn = jnp.zeros((_L,), jnp.float32)
                sq = jnp.zeros((_L,), jnp.float32)
                for j in range(_NSTRIP):
                    sl = pl.ds(_L * j, _L)
                    pc = pc + urows[b, sl] * embv[j]
                    pn = pn + nrows[b, sl] * embv[j]
                    dd = vspu[b, sl] - vspi[b, sl]
                    sq = sq + dd * dd
                outc_s[pl.ds(b * _L, _L)] = pc
                outn_s[pl.ds(b * _L, _L)] = pn
                vacc[...] = vacc[...] + sq
                return 0

            lax.fori_loop(0, _NB, b_body, 0)
            out_copies(g, p, issue=True)

        copies(0, 0, issue=True)

        def pair_body(i, _):
            g = 2 * i
            copies(g + 1, 1, issue=True)
            copies(g, 0, issue=False)
            compute(g, 0)

            @pl.when(g + 2 < nch)
            def _():
                copies(g + 2, 0, issue=True)

            copies(g + 1, 1, issue=False)
            compute(g + 1, 1)
            return 0

        lax.fori_loop(0, nch // 2, pair_body, 0)
        out_copies(nch - 2, 0, issue=False)
        out_copies(nch - 1, 1, issue=False)
        pltpu.sync_copy(vacc, outvsp_h.at[pl.ds(wid * _L, _L)])

    return k(batch_u, batch_n, batch_v_pad, batch_v_mask, batch_vsp,
             W_i, W_u, W_v)


def _tc_loss_body(pc_ref, pn_ref, vsp_ref, out_ref):
    sc = jnp.sum(pc_ref[...], axis=1)
    sn = jnp.sum(pn_ref[...], axis=1)

    def log_sigmoid(x):
        return jnp.minimum(x, 0.0) - jnp.log(1.0 + jnp.exp(-jnp.abs(x)))

    score1 = jnp.sum(jnp.maximum(_MARGIN + log_sigmoid(sn) - log_sigmoid(sc),
                                 0.0))
    out_ref[0, 0] = score1 + _SIGMA * jnp.sum(vsp_ref[...])


def kernel(batch_u, batch_n, batch_v_pad, batch_v_mask, batch_vsp,
           W_i, W_u, W_v):
    out_c, out_n, out_vsp = _sc_gather_scores(
        batch_u, batch_n, batch_v_pad, batch_v_mask, batch_vsp, W_i, W_u, W_v)
    out_c = out_c.reshape(_B, _L)
    out_n = out_n.reshape(_B, _L)
    out_vsp = out_vsp.reshape(-1, _L)
    loss = pl.pallas_call(
        _tc_loss_body,
        out_shape=jax.ShapeDtypeStruct((1, 1), jnp.float32),
        out_specs=pl.BlockSpec(memory_space=pltpu.SMEM),
    )(out_c, out_n, out_vsp)
    return loss[0, 0]
